# baseline (speedup 1.0000x reference)
import numpy as np
import jax
import jax.numpy as jnp
from jax.experimental import pallas as pl

B, N = 16, 64
NF = 128
RW = 8
L = 4
DEG_MAX = 32
XCH, PCH, ECH = NF, NF // 2, NF // 2
G = 2
KA = N + RW + 3


def _act(v):
    w = 0.5 * v
    return w * (jnp.tanh(w) + 1.0)


def _hswish(w):
    return w * (jnp.tanh(w) + 1.0)


def _prologue_body(tc_ref, w1_ref, b1_ref, w2_ref, b2_ref, tp_ref, tpb_ref,
                   degb_ref, oriw_ref, orib_ref, einw_ref, einb_ref,
                   spdw_ref, spdb_ref, msgw_ref, msgb_ref,
                   e1_ref, e2_ref, eob_ref, co2w_ref, posw_ref,
                   updw_ref, updb_ref, co1w_ref, co1b_ref,
                   tvec_ref, mu_ref, mT_ref, mc_ref, w1f_ref, T2_ref,
                   cfin_ref, co2r_ref, pwb_ref, uwh_ref, ubh_ref,
                   co1h_ref, co1bh_ref):
    dot = lambda a, b: jnp.dot(a, b, preferred_element_type=jnp.float32)
    half = NF // 2
    fi = jax.lax.broadcasted_iota(jnp.int32, (1, half), 1).astype(jnp.float32)
    freqs = jnp.exp((-np.log(10000.0) / (half - 1)) * fi)
    args = tc_ref[...] * freqs
    emb = jnp.concatenate([jnp.sin(args), jnp.cos(args)], axis=1)
    t = dot(emb, w1_ref[...]) + b1_ref[...]
    t = dot(_act(t), w2_ref[...]) + b2_ref[...]
    tvec_ref[:, 0, :] = dot(_act(t), tp_ref[...]) + tpb_ref[...] + degb_ref[...]

    ein1 = einw_ref[:ECH, :]
    ein2 = einw_ref[ECH:, :]
    A = dot(oriw_ref[...], ein1)
    C0 = dot(orib_ref[...], ein1) + einb_ref[...]
    spd_tab = dot(spdw_ref[...] + spdb_ref[...], ein2)
    for l in range(L):
        we = msgw_ref[l, XCH + PCH:, :]
        mu_ref[l:l + 1, :] = dot(A, we)
        mT_ref[l] = dot(spd_tab, we)
        mc_ref[l:l + 1, :] = dot(C0, we) + msgb_ref[l:l + 1, :]
    w1f_ref[...] = dot(oriw_ref[...], e1_ref[...])
    T2_ref[...] = dot(spdw_ref[...], e2_ref[...])
    cfin_ref[...] = dot(orib_ref[...], e1_ref[...]) + dot(spdb_ref[...], e2_ref[...]) + eob_ref[...]
    co2r_ref[...] = co2w_ref[...].T
    pwb_ref[...] = jnp.broadcast_to(posw_ref[...][:, None, :], (RW, N, PCH))
    uwh_ref[...] = 0.5 * updw_ref[...]
    ubh_ref[...] = 0.5 * updb_ref[...]
    co1h_ref[...] = 0.5 * co1w_ref[...]
    co1bh_ref[...] = 0.5 * co1b_ref[...]


def _main_body(x_ref, m_ref, tvec_ref, degw_ref, pwb_ref, posb_ref,
               msgw_ref, mc_ref, mu_ref, mT_ref, uwh_ref, ubh_ref,
               eos_ref, eod_ref, w1f_ref, T2_ref, cfin_ref,
               co1h_ref, co1bh_ref, co2_ref, co2b_ref, out_ref):
    dot = lambda a, b: jnp.dot(a, b, preferred_element_type=jnp.float32)
    ii = jax.lax.broadcasted_iota(jnp.int32, (N, N), 0)
    jj = jax.lax.broadcasted_iota(jnp.int32, (N, N), 1)
    diag_m = (ii == jj).astype(jnp.float32)
    kdeg = jax.lax.broadcasted_iota(jnp.int32, (N, DEG_MAX + 1), 1).astype(jnp.float32)
    La = jax.lax.broadcasted_iota(jnp.int32, (N, N, KA), 2)
    ja = jax.lax.broadcasted_iota(jnp.int32, (N, N, KA), 1)
    AmJ = La == ja
    AmX = La == N + RW + 1
    AmE = La == N + RW + 2

    for g in range(G):
        xr = x_ref[g, 0]
        mk = m_ref[g, 0]
        xc = jnp.clip(jnp.where(jnp.isnan(xr), 0.0, xr), -1.0, 1.0)
        cont = jnp.clip((xc + 1.0) * 0.5 * mk, 0.0, 1.0)
        cont = jnp.where(cont < 0.5, 0.0, cont)
        emask = (cont > 0.0).astype(jnp.float32)
        adjb = jnp.where(xc >= 0.0, 1.0, 0.0) * mk
        degb = adjb.sum(-1, keepdims=True)
        AD = adjb / (degb + 1e-8)

        pos = jnp.zeros((N, PCH), jnp.float32)
        zcnt = jnp.zeros((N, N), jnp.float32)
        P = AD
        for k in range(RW):
            P = dot(P, AD)
            pos = pos + dot(P * diag_m, pwb_ref[k])
            zcnt = zcnt + (P <= 0.0).astype(jnp.float32)
        pos = pos + posb_ref[...]

        sz = zcnt.astype(jnp.int32) + N
        Am3 = (AmJ | (La == sz[:, :, None])).astype(jnp.float32)
        Am3 = Am3 + jnp.where(AmX, xc[:, :, None], 0.0)
        AmM3 = Am3 + jnp.where(AmE, emask[:, :, None] - 1.0, 0.0)
        Am = Am3.reshape(N * N, KA)
        AmM = AmM3.reshape(N * N, KA)

        degf = jnp.floor(jnp.clip(cont.sum(-1), 0.0, float(DEG_MAX)))
        degOH = (kdeg == degf[:, None]).astype(jnp.float32)
        h = dot(degOH, degw_ref[...]) + tvec_ref[g]

        for l in range(L):
            hp = jnp.concatenate([h, pos], axis=-1)
            nodep = dot(hp, msgw_ref[l, :XCH + PCH, :]) + mc_ref[l:l + 1, :]
            W = jnp.concatenate([nodep, mT_ref[l], mu_ref[l:l + 1, :],
                                 jnp.full((1, XCH), 20.0, jnp.float32)], axis=0) * 0.5
            Eh = dot(AmM, W).reshape(N, N, XCH)
            m = _hswish(Eh)
            agg = m.sum(axis=1)
            h = h + _hswish(dot(agg, uwh_ref[l]) + ubh_ref[l:l + 1, :])
            mu = h.mean(-1, keepdims=True)
            var = ((h - mu) ** 2).mean(-1, keepdims=True)
            h = (h - mu) / jnp.sqrt(var + 1e-5)

        nf = jnp.concatenate([h, pos], axis=-1)
        a_src = (dot(nf, eos_ref[...]) + cfin_ref[...]) * 0.5
        a_dst = dot(nf, eod_ref[...])
        W2 = jnp.concatenate([a_dst, T2_ref[...], w1f_ref[...],
                              jnp.zeros((1, ECH), jnp.float32)], axis=0) * 0.5
        edh = dot(Am, W2).reshape(N, N, ECH) + a_src[:, None, :]
        v1 = _hswish(edh)
        gg = _hswish(0.5 * v1).reshape(N * N, ECH)
        o1 = _hswish(dot(gg, co1h_ref[...]) + co1bh_ref[...])
        om = (o1.reshape(N, N, ECH) * co2_ref[...][None, :, :]).sum(-1) + co2b_ref[0, 0]
        out_ref[g, 0] = (om + om.T) * 0.5 * mk


def kernel(x, time_cond, mask, params):
    p = params
    r2 = lambda v: v[None, :]
    f32 = jnp.float32
    (tvec, mu_l, mT, mc, w1f, T2, cfin, co2row, pwb,
     uwh, ubh, co1h, co1bh) = pl.pallas_call(
        _prologue_body,
        out_shape=[
            jax.ShapeDtypeStruct((B, 1, XCH), f32),
            jax.ShapeDtypeStruct((L, XCH), f32),
            jax.ShapeDtypeStruct((L, RW + 1, XCH), f32),
            jax.ShapeDtypeStruct((L, XCH), f32),
            jax.ShapeDtypeStruct((1, ECH), f32),
            jax.ShapeDtypeStruct((RW + 1, ECH), f32),
            jax.ShapeDtypeStruct((1, ECH), f32),
            jax.ShapeDtypeStruct((1, ECH), f32),
            jax.ShapeDtypeStruct((RW, N, PCH), f32),
            jax.ShapeDtypeStruct((L, XCH, XCH), f32),
            jax.ShapeDtypeStruct((L, XCH), f32),
            jax.ShapeDtypeStruct((ECH, ECH), f32),
            jax.ShapeDtypeStruct((1, ECH), f32),
        ],
    )(time_cond.reshape(B, 1), p["temb_w1"], r2(p["temb_b1"]), p["temb_w2"],
      r2(p["temb_b2"]), p["tproj_w"], r2(p["tproj_b"]), r2(p["deg_b"]),
      p["ori_w"], r2(p["ori_b"]), p["ein_w"], r2(p["ein_b"]),
      p["spd_w"], r2(p["spd_b"]), p["msg_w"], p["msg_b"],
      p["eo_e1"], p["eo_e2"], r2(p["eo_b"]), p["co2_w"], p["pos_w"],
      p["upd_w"], p["upd_b"], p["co1_w"], r2(p["co1_b"]))

    full = lambda *shape: pl.BlockSpec(shape, lambda b: (0,) * len(shape))
    out = pl.pallas_call(
        _main_body,
        grid=(B // G,),
        in_specs=[
            pl.BlockSpec((G, 1, N, N), lambda b: (b, 0, 0, 0)),
            pl.BlockSpec((G, 1, N, N), lambda b: (b, 0, 0, 0)),
            pl.BlockSpec((G, 1, XCH), lambda b: (b, 0, 0)),
            full(DEG_MAX + 1, XCH),
            full(RW, N, PCH),
            full(1, PCH),
            full(L, 2 * XCH, XCH),
            full(L, XCH),
            full(L, XCH),
            full(L, RW + 1, XCH),
            full(L, XCH, XCH),
            full(L, XCH),
            full(XCH + PCH, ECH),
            full(XCH + PCH, ECH),
            full(1, ECH),
            full(RW + 1, ECH),
            full(1, ECH),
            full(ECH, ECH),
            full(1, ECH),
            full(1, ECH),
            full(1, 1),
        ],
        out_specs=pl.BlockSpec((G, 1, N, N), lambda b: (b, 0, 0, 0)),
        out_shape=jax.ShapeDtypeStruct((B, 1, N, N), f32),
    )(x, mask, tvec, p["deg_w"], pwb, r2(p["pos_b"]),
      p["msg_w"], mc, mu_l, mT, uwh, ubh,
      p["eo_src"], p["eo_dst"], w1f, T2, cfin,
      co1h, co1bh, co2row, p["co2_b"].reshape(1, 1))
    return out


# speedup vs baseline: 1.0309x; 1.0309x over previous
import numpy as np
import jax
import jax.numpy as jnp
from jax.experimental import pallas as pl

B, N = 16, 64
NF = 128
RW = 8
L = 4
DEG_MAX = 32
XCH, PCH, ECH = NF, NF // 2, NF // 2
G = 2
KA = N + RW + 2


def _act(v):
    w = 0.5 * v
    return w * (jnp.tanh(w) + 1.0)


def _hswish(w):
    return w * (jnp.tanh(w) + 1.0)


def _prologue_body(tc_ref, w1_ref, b1_ref, w2_ref, b2_ref, tp_ref, tpb_ref,
                   degb_ref, oriw_ref, orib_ref, einw_ref, einb_ref,
                   spdw_ref, spdb_ref, msgw_ref, msgb_ref,
                   e1_ref, e2_ref, eob_ref, co2w_ref, posw_ref,
                   updw_ref, updb_ref, co1w_ref, co1b_ref,
                   tvec_ref, mu_ref, mT_ref, mc_ref, w1f_ref, T2_ref,
                   cfin_ref, co2r_ref, pwb_ref, uwh_ref, ubh_ref,
                   co1h_ref, co1bh_ref):
    dot = lambda a, b: jnp.dot(a, b, preferred_element_type=jnp.float32)
    half = NF // 2
    fi = jax.lax.broadcasted_iota(jnp.int32, (1, half), 1).astype(jnp.float32)
    freqs = jnp.exp((-np.log(10000.0) / (half - 1)) * fi)
    args = tc_ref[...] * freqs
    emb = jnp.concatenate([jnp.sin(args), jnp.cos(args)], axis=1)
    t = dot(emb, w1_ref[...]) + b1_ref[...]
    t = dot(_act(t), w2_ref[...]) + b2_ref[...]
    tvec_ref[:, 0, :] = dot(_act(t), tp_ref[...]) + tpb_ref[...] + degb_ref[...]

    ein1 = einw_ref[:ECH, :]
    ein2 = einw_ref[ECH:, :]
    A = dot(oriw_ref[...], ein1)
    C0 = dot(orib_ref[...], ein1) + einb_ref[...]
    spd_tab = dot(spdw_ref[...] + spdb_ref[...], ein2)
    for l in range(L):
        we = msgw_ref[l, XCH + PCH:, :]
        mu_ref[l:l + 1, :] = dot(A, we)
        mT_ref[l] = dot(spd_tab, we)
        mc_ref[l:l + 1, :] = dot(C0, we) + msgb_ref[l:l + 1, :]
    w1f_ref[...] = dot(oriw_ref[...], e1_ref[...])
    T2_ref[...] = dot(spdw_ref[...], e2_ref[...])
    cfin_ref[...] = dot(orib_ref[...], e1_ref[...]) + dot(spdb_ref[...], e2_ref[...]) + eob_ref[...]
    co2r_ref[...] = co2w_ref[...].T
    pwb_ref[...] = jnp.broadcast_to(posw_ref[...][:, None, :], (RW, N, PCH))
    uwh_ref[...] = 0.5 * updw_ref[...]
    ubh_ref[...] = 0.5 * updb_ref[...]
    co1h_ref[...] = 0.5 * co1w_ref[...]
    co1bh_ref[...] = 0.5 * co1b_ref[...]


def _main_body(x_ref, m_ref, tvec_ref, degw_ref, pwb_ref, posb_ref,
               msgw_ref, mc_ref, mu_ref, mT_ref, uwh_ref, ubh_ref,
               eos_ref, eod_ref, w1f_ref, T2_ref, cfin_ref,
               co1h_ref, co1bh_ref, co2_ref, co2b_ref, out_ref):
    dot = lambda a, b: jnp.dot(a, b, preferred_element_type=jnp.float32)
    ii = jax.lax.broadcasted_iota(jnp.int32, (N, N), 0)
    jj = jax.lax.broadcasted_iota(jnp.int32, (N, N), 1)
    diag_m = (ii == jj).astype(jnp.float32)
    kdeg = jax.lax.broadcasted_iota(jnp.int32, (N, DEG_MAX + 1), 1).astype(jnp.float32)
    La = jax.lax.broadcasted_iota(jnp.int32, (N, N, KA), 2)
    ja = jax.lax.broadcasted_iota(jnp.int32, (N, N, KA), 1)
    AmJ = La == ja
    AmX = La == N + RW + 1

    for g in range(G):
        xr = x_ref[g, 0]
        mk = m_ref[g, 0]
        xc = jnp.clip(jnp.where(jnp.isnan(xr), 0.0, xr), -1.0, 1.0)
        cont = jnp.clip((xc + 1.0) * 0.5 * mk, 0.0, 1.0)
        cont = jnp.where(cont < 0.5, 0.0, cont)
        emask = (cont > 0.0).astype(jnp.float32)
        adjb = jnp.where(xc >= 0.0, 1.0, 0.0) * mk
        degb = adjb.sum(-1, keepdims=True)
        AD = adjb / (degb + 1e-8)

        pos = jnp.zeros((N, PCH), jnp.float32)
        zcnt = jnp.zeros((N, N), jnp.float32)
        P = AD
        for k in range(RW):
            P = dot(P, AD)
            pos = pos + dot(P * diag_m, pwb_ref[k])
            zcnt = zcnt + (P <= 0.0).astype(jnp.float32)
        pos = pos + posb_ref[...]

        sz = zcnt.astype(jnp.int32) + N
        Am3 = (AmJ | (La == sz[:, :, None])).astype(jnp.float32)
        Am3 = Am3 + jnp.where(AmX, xc[:, :, None], 0.0)
        Am = Am3.reshape(N * N, KA)
        emask3 = emask[:, :, None]

        degf = jnp.floor(jnp.clip(cont.sum(-1), 0.0, float(DEG_MAX)))
        degOH = (kdeg == degf[:, None]).astype(jnp.float32)
        h = dot(degOH, degw_ref[...]) + tvec_ref[g]

        for l in range(L):
            hp = jnp.concatenate([h, pos], axis=-1)
            nodep = dot(hp, msgw_ref[l, :XCH + PCH, :]) + mc_ref[l:l + 1, :]
            W = jnp.concatenate([nodep, mT_ref[l], mu_ref[l:l + 1, :]], axis=0) * 0.5
            Eh = dot(Am, W).reshape(N, N, XCH)
            m = (Eh * emask3) * (jnp.tanh(Eh) + 1.0)
            agg = m.sum(axis=1)
            h = h + _hswish(dot(agg, uwh_ref[l]) + ubh_ref[l:l + 1, :])
            mu = h.mean(-1, keepdims=True)
            var = ((h - mu) ** 2).mean(-1, keepdims=True)
            h = (h - mu) * jax.lax.rsqrt(var + 1e-5)

        nf = jnp.concatenate([h, pos], axis=-1)
        a_src = (dot(nf, eos_ref[...]) + cfin_ref[...]) * 0.5
        a_dst = dot(nf, eod_ref[...])
        W2 = jnp.concatenate([a_dst, T2_ref[...], w1f_ref[...]], axis=0) * 0.5
        edh = dot(Am, W2).reshape(N, N, ECH) + a_src[:, None, :]
        v1 = _hswish(edh)
        gg = _hswish(0.5 * v1).reshape(N * N, ECH)
        o1 = _hswish(dot(gg, co1h_ref[...]) + co1bh_ref[...])
        om = (o1.reshape(N, N, ECH) * co2_ref[...][None, :, :]).sum(-1) + co2b_ref[0, 0]
        out_ref[g, 0] = (om + om.T) * 0.5 * mk


def kernel(x, time_cond, mask, params):
    p = params
    r2 = lambda v: v[None, :]
    f32 = jnp.float32
    (tvec, mu_l, mT, mc, w1f, T2, cfin, co2row, pwb,
     uwh, ubh, co1h, co1bh) = pl.pallas_call(
        _prologue_body,
        out_shape=[
            jax.ShapeDtypeStruct((B, 1, XCH), f32),
            jax.ShapeDtypeStruct((L, XCH), f32),
            jax.ShapeDtypeStruct((L, RW + 1, XCH), f32),
            jax.ShapeDtypeStruct((L, XCH), f32),
            jax.ShapeDtypeStruct((1, ECH), f32),
            jax.ShapeDtypeStruct((RW + 1, ECH), f32),
            jax.ShapeDtypeStruct((1, ECH), f32),
            jax.ShapeDtypeStruct((1, ECH), f32),
            jax.ShapeDtypeStruct((RW, N, PCH), f32),
            jax.ShapeDtypeStruct((L, XCH, XCH), f32),
            jax.ShapeDtypeStruct((L, XCH), f32),
            jax.ShapeDtypeStruct((ECH, ECH), f32),
            jax.ShapeDtypeStruct((1, ECH), f32),
        ],
    )(time_cond.reshape(B, 1), p["temb_w1"], r2(p["temb_b1"]), p["temb_w2"],
      r2(p["temb_b2"]), p["tproj_w"], r2(p["tproj_b"]), r2(p["deg_b"]),
      p["ori_w"], r2(p["ori_b"]), p["ein_w"], r2(p["ein_b"]),
      p["spd_w"], r2(p["spd_b"]), p["msg_w"], p["msg_b"],
      p["eo_e1"], p["eo_e2"], r2(p["eo_b"]), p["co2_w"], p["pos_w"],
      p["upd_w"], p["upd_b"], p["co1_w"], r2(p["co1_b"]))

    full = lambda *shape: pl.BlockSpec(shape, lambda b: (0,) * len(shape))
    out = pl.pallas_call(
        _main_body,
        grid=(B // G,),
        in_specs=[
            pl.BlockSpec((G, 1, N, N), lambda b: (b, 0, 0, 0)),
            pl.BlockSpec((G, 1, N, N), lambda b: (b, 0, 0, 0)),
            pl.BlockSpec((G, 1, XCH), lambda b: (b, 0, 0)),
            full(DEG_MAX + 1, XCH),
            full(RW, N, PCH),
            full(1, PCH),
            full(L, 2 * XCH, XCH),
            full(L, XCH),
            full(L, XCH),
            full(L, RW + 1, XCH),
            full(L, XCH, XCH),
            full(L, XCH),
            full(XCH + PCH, ECH),
            full(XCH + PCH, ECH),
            full(1, ECH),
            full(RW + 1, ECH),
            full(1, ECH),
            full(ECH, ECH),
            full(1, ECH),
            full(1, ECH),
            full(1, 1),
        ],
        out_specs=pl.BlockSpec((G, 1, N, N), lambda b: (b, 0, 0, 0)),
        out_shape=jax.ShapeDtypeStruct((B, 1, N, N), f32),
    )(x, mask, tvec, p["deg_w"], pwb, r2(p["pos_b"]),
      p["msg_w"], mc, mu_l, mT, uwh, ubh,
      p["eo_src"], p["eo_dst"], w1f, T2, cfin,
      co1h, co1bh, co2row, p["co2_b"].reshape(1, 1))
    return out
